# baseline (speedup 1.0000x reference)
import functools

import jax
import jax.numpy as jnp
from jax import lax
from jax.experimental import pallas as pl
from jax.experimental.pallas import tpu as pltpu
from jax.experimental.pallas import tpu_sc as plsc

N = 10000
D = 128
H = 128
C = 64
E = 320000
W = 64

NC = 2
NS = 16
CHUNK = 128
G = 80
GP = 80
NCHUNKS = NC * NS * G
NBUF = 5
E_PAD = NCHUNKS * CHUNK
N_PAD = 10112
RPT = N_PAD // NS


@functools.lru_cache(maxsize=None)
def _seg_sum_kernel():
    mesh = plsc.VectorSubcoreMesh(core_axis_name="c", subcore_axis_name="s")

    @functools.partial(
        pl.kernel,
        out_type=jax.ShapeDtypeStruct((NC, N_PAD, W), jnp.float32),
        mesh=mesh,
        compiler_params=pltpu.CompilerParams(use_tc_tiling_on_sc=False),
        scratch_types=[
            [pltpu.VMEM((GP, CHUNK), jnp.int32) for _ in range(G // GP)],
            [pltpu.VMEM((GP, CHUNK), jnp.int32) for _ in range(G // GP)],
            [pltpu.VMEM((CHUNK, W), jnp.float32) for _ in range(NBUF)],
            pltpu.VMEM_SHARED((N_PAD, W), jnp.float32),
            [pltpu.SemaphoreType.DMA for _ in range(NBUF)],
            [pltpu.SemaphoreType.DMA for _ in range(NBUF)],
        ],
    )
    def seg_sum(x_hbm, src_hbm, dst_hbm, zeros_hbm, out_hbm,
                src_v, dst_v, bufs, agg_sh, gsems, ssems):
        c = lax.axis_index("c")
        s = lax.axis_index("s")
        wid = c * NS + s
        for ph in range(G // GP):
            pltpu.sync_copy(src_hbm.at[pl.ds(wid * G + ph * GP, GP)], src_v[ph])
            pltpu.sync_copy(dst_hbm.at[pl.ds(wid * G + ph * GP, GP)], dst_v[ph])
        pltpu.sync_copy(zeros_hbm, agg_sh.at[pl.ds(s * RPT, RPT)])
        plsc.subcore_barrier()

        for ph in range(G // GP):
            sv, dv = src_v[ph], dst_v[ph]
            for j in range(NBUF):
                pltpu.async_copy(x_hbm.at[sv.at[j]], bufs[j], gsems[j])

            def body(k, carry, sv=sv, dv=dv):
                for j in range(NBUF):
                    g = k * NBUF + j
                    pltpu.make_async_copy(
                        x_hbm.at[sv.at[g]], bufs[j], gsems[j]).wait()
                    pltpu.async_copy(
                        bufs[j], agg_sh.at[dv.at[g]], ssems[j], add=True)
                    pltpu.make_async_copy(
                        bufs[j], agg_sh.at[dv.at[g]], ssems[j]).wait()

                    @pl.when(k < GP // NBUF - 1)
                    def _():
                        pltpu.async_copy(
                            x_hbm.at[sv.at[g + NBUF]], bufs[j], gsems[j])
                return carry

            lax.fori_loop(0, GP // NBUF, body, 0)
        plsc.subcore_barrier()
        pltpu.sync_copy(agg_sh.at[pl.ds(s * RPT, RPT)],
                        out_hbm.at[c].at[pl.ds(s * RPT, RPT)])

    return seg_sum


BM = 1000


def _l1_body(x_ref, pl_ref, pr_ref, w1_ref, b1_ref, w2_ref, o_ref):
    agg = jnp.concatenate(
        [pl_ref[0] + pl_ref[1], pr_ref[0] + pr_ref[1]], axis=-1)
    h = jnp.dot(x_ref[...] + agg, w1_ref[...],
                preferred_element_type=jnp.float32) + b1_ref[...]
    o_ref[...] = jnp.dot(jnp.maximum(h, 0.0), w2_ref[...],
                         preferred_element_type=jnp.float32)


def _l2_body(y_ref, q_ref, b2_ref, o_ref):
    o_ref[...] = y_ref[...] + q_ref[0] + q_ref[1] + b2_ref[...]


@functools.lru_cache(maxsize=None)
def _tc_layer1():
    return pl.pallas_call(
        _l1_body,
        grid=(N // BM,),
        in_specs=[
            pl.BlockSpec((BM, D), lambda i: (i, 0)),
            pl.BlockSpec((NC, BM, W), lambda i: (0, i, 0)),
            pl.BlockSpec((NC, BM, W), lambda i: (0, i, 0)),
            pl.BlockSpec((D, H), lambda i: (0, 0)),
            pl.BlockSpec((1, H), lambda i: (0, 0)),
            pl.BlockSpec((H, C), lambda i: (0, 0)),
        ],
        out_specs=pl.BlockSpec((BM, C), lambda i: (i, 0)),
        out_shape=jax.ShapeDtypeStruct((N, C), jnp.float32),
    )


@functools.lru_cache(maxsize=None)
def _tc_layer2():
    return pl.pallas_call(
        _l2_body,
        grid=(N // BM,),
        in_specs=[
            pl.BlockSpec((BM, C), lambda i: (i, 0)),
            pl.BlockSpec((NC, BM, C), lambda i: (0, i, 0)),
            pl.BlockSpec((1, C), lambda i: (0, 0)),
        ],
        out_specs=pl.BlockSpec((BM, C), lambda i: (i, 0)),
        out_shape=jax.ShapeDtypeStruct((N, C), jnp.float32),
    )


def kernel(feats, edge_index, W1, b1, W2, b2):
    src = edge_index[0].astype(jnp.int32)
    dst = edge_index[1].astype(jnp.int32)
    pad_i = jnp.arange(E_PAD - E, dtype=jnp.int32)
    src_p = jnp.concatenate(
        [src, pad_i % N]).reshape(NCHUNKS, CHUNK)
    dst_p = jnp.concatenate(
        [dst, N + pad_i % (N_PAD - N)]).reshape(NCHUNKS, CHUNK)
    zeros = jnp.zeros((RPT, W), jnp.float32)

    seg_sum = _seg_sum_kernel()
    p_l = seg_sum(feats[:, :W], src_p, dst_p, zeros)
    p_r = seg_sum(feats[:, W:], src_p, dst_p, zeros)
    y2 = _tc_layer1()(feats, p_l, p_r, W1, b1.reshape(1, H), W2)
    q = seg_sum(y2, src_p, dst_p, zeros)
    out = _tc_layer2()(y2, q, b2.reshape(1, C))
    return out


# speedup vs baseline: 1.0206x; 1.0206x over previous
import functools

import jax
import jax.numpy as jnp
from jax import lax
from jax.experimental import pallas as pl
from jax.experimental.pallas import tpu as pltpu
from jax.experimental.pallas import tpu_sc as plsc

N = 10000
D = 128
H = 128
C = 64
E = 320000
W = 64

NC = 2
NS = 16
CHUNK = 128
G = 80
GP = 80
NCHUNKS = NC * NS * G
NBUF = 4
E_PAD = NCHUNKS * CHUNK
N_PAD = 10112
RPT = N_PAD // NS


@functools.lru_cache(maxsize=None)
def _seg_sum_kernel():
    mesh = plsc.VectorSubcoreMesh(core_axis_name="c", subcore_axis_name="s")

    @functools.partial(
        pl.kernel,
        out_type=jax.ShapeDtypeStruct((NC, N_PAD, W), jnp.float32),
        mesh=mesh,
        compiler_params=pltpu.CompilerParams(use_tc_tiling_on_sc=False),
        scratch_types=[
            [pltpu.VMEM((GP, CHUNK), jnp.int32) for _ in range(G // GP)],
            [pltpu.VMEM((GP, CHUNK), jnp.int32) for _ in range(G // GP)],
            [pltpu.VMEM((CHUNK, W), jnp.float32) for _ in range(NBUF)],
            pltpu.VMEM_SHARED((N_PAD, W), jnp.float32),
            [pltpu.SemaphoreType.DMA for _ in range(NBUF)],
            [pltpu.SemaphoreType.DMA for _ in range(NBUF)],
        ],
    )
    def seg_sum(x_hbm, src_hbm, dst_hbm, zeros_hbm, out_hbm,
                src_v, dst_v, bufs, agg_sh, gsems, ssems):
        c = lax.axis_index("c")
        s = lax.axis_index("s")
        wid = c * NS + s
        for ph in range(G // GP):
            pltpu.sync_copy(src_hbm.at[pl.ds(wid * G + ph * GP, GP)], src_v[ph])
            pltpu.sync_copy(dst_hbm.at[pl.ds(wid * G + ph * GP, GP)], dst_v[ph])
        pltpu.sync_copy(zeros_hbm, agg_sh.at[pl.ds(s * RPT, RPT)])
        plsc.subcore_barrier()

        for ph in range(G // GP):
            sv, dv = src_v[ph], dst_v[ph]
            for j in range(NBUF):
                pltpu.async_copy(x_hbm.at[sv.at[j]], bufs[j], gsems[j])

            def body(k, carry, sv=sv, dv=dv):
                for j in range(NBUF):
                    g = k * NBUF + j
                    pltpu.make_async_copy(
                        x_hbm.at[sv.at[g]], bufs[j], gsems[j]).wait()
                    pltpu.async_copy(
                        bufs[j], agg_sh.at[dv.at[g]], ssems[j], add=True)
                    pltpu.make_async_copy(
                        bufs[j], agg_sh.at[dv.at[g]], ssems[j]).wait()

                    @pl.when(k < GP // NBUF - 1)
                    def _():
                        pltpu.async_copy(
                            x_hbm.at[sv.at[g + NBUF]], bufs[j], gsems[j])
                return carry

            lax.fori_loop(0, GP // NBUF, body, 0)
        plsc.subcore_barrier()
        pltpu.sync_copy(agg_sh.at[pl.ds(s * RPT, RPT)],
                        out_hbm.at[c].at[pl.ds(s * RPT, RPT)])

    return seg_sum


BM = 2000


def _l1_body(x_ref, pl_ref, pr_ref, w1_ref, b1_ref, w2_ref, o_ref):
    agg = jnp.concatenate(
        [pl_ref[0] + pl_ref[1], pr_ref[0] + pr_ref[1]], axis=-1)
    h = jnp.dot(x_ref[...] + agg, w1_ref[...],
                preferred_element_type=jnp.float32) + b1_ref[...]
    o_ref[...] = jnp.dot(jnp.maximum(h, 0.0), w2_ref[...],
                         preferred_element_type=jnp.float32)


def _l2_body(y_ref, q_ref, b2_ref, o_ref):
    o_ref[...] = y_ref[...] + q_ref[0] + q_ref[1] + b2_ref[...]


@functools.lru_cache(maxsize=None)
def _tc_layer1():
    return pl.pallas_call(
        _l1_body,
        grid=(N // BM,),
        in_specs=[
            pl.BlockSpec((BM, D), lambda i: (i, 0)),
            pl.BlockSpec((NC, BM, W), lambda i: (0, i, 0)),
            pl.BlockSpec((NC, BM, W), lambda i: (0, i, 0)),
            pl.BlockSpec((D, H), lambda i: (0, 0)),
            pl.BlockSpec((1, H), lambda i: (0, 0)),
            pl.BlockSpec((H, C), lambda i: (0, 0)),
        ],
        out_specs=pl.BlockSpec((BM, C), lambda i: (i, 0)),
        out_shape=jax.ShapeDtypeStruct((N, C), jnp.float32),
    )


@functools.lru_cache(maxsize=None)
def _tc_layer2():
    return pl.pallas_call(
        _l2_body,
        grid=(N // BM,),
        in_specs=[
            pl.BlockSpec((BM, C), lambda i: (i, 0)),
            pl.BlockSpec((NC, BM, C), lambda i: (0, i, 0)),
            pl.BlockSpec((1, C), lambda i: (0, 0)),
        ],
        out_specs=pl.BlockSpec((BM, C), lambda i: (i, 0)),
        out_shape=jax.ShapeDtypeStruct((N, C), jnp.float32),
    )


def kernel(feats, edge_index, W1, b1, W2, b2):
    src = edge_index[0].astype(jnp.int32)
    dst = edge_index[1].astype(jnp.int32)
    pad_i = jnp.arange(E_PAD - E, dtype=jnp.int32)
    src_p = jnp.concatenate(
        [src, pad_i % N]).reshape(NCHUNKS, CHUNK)
    dst_p = jnp.concatenate(
        [dst, N + pad_i % (N_PAD - N)]).reshape(NCHUNKS, CHUNK)
    zeros = jnp.zeros((RPT, W), jnp.float32)

    seg_sum = _seg_sum_kernel()
    p_l = seg_sum(feats[:, :W], src_p, dst_p, zeros)
    p_r = seg_sum(feats[:, W:], src_p, dst_p, zeros)
    y2 = _tc_layer1()(feats, p_l, p_r, W1, b1.reshape(1, H), W2)
    q = seg_sum(y2, src_p, dst_p, zeros)
    out = _tc_layer2()(y2, q, b2.reshape(1, C))
    return out
